# baseline (speedup 1.0000x reference)
import jax
import jax.numpy as jnp
from jax.experimental import pallas as pl
from jax.experimental.pallas import tpu as pltpu

_N_LEVELS = 2.0 ** 8 - 1.0
_NA = 8
_NB = 8


def _body(in_hbm, out_hbm, buf, ostage, isem, osem):
    rows, cols = buf.shape
    ra = rows // _NA
    rb = rows // _NB

    def copy_in(i, k):
        return pltpu.make_async_copy(
            in_hbm.at[pl.ds(i * ra, ra), :],
            buf.at[pl.ds(i * ra, ra), :],
            isem.at[k])

    def copy_out(i, k):
        return pltpu.make_async_copy(
            ostage.at[pl.ds(pl.multiple_of(k * rb, rb), rb), :],
            out_hbm.at[pl.ds(i * rb, rb), :],
            osem.at[k])

    for j in range(_NA):
        copy_in(j, j).start()

    def phase_a(i, m):
        copy_in(i, i).wait()
        band = buf[pl.ds(i * ra, ra), :]
        return jnp.maximum(m, jnp.max(jnp.abs(band)))

    alpha = jax.lax.fori_loop(0, _NA, phase_a, jnp.float32(0.0))

    d = jnp.maximum(2.0 * alpha, 1e-8)
    scale = _N_LEVELS / d
    zp = scale * (-alpha)
    inv = d * (1.0 / _N_LEVELS)

    def phase_b(i, carry):
        k = i % 2

        @pl.when(i >= 2)
        def _():
            copy_out(i - 2, k).wait()

        x = buf[pl.ds(i * rb, rb), :]
        q = jnp.round(x * scale - zp)
        ostage[pl.ds(pl.multiple_of(k * rb, rb), rb), :] = (q + zp) * inv
        copy_out(i, k).start()
        return carry

    jax.lax.fori_loop(0, _NB, phase_b, 0)
    copy_out(_NB - 2, (_NB - 2) % 2).wait()
    copy_out(_NB - 1, (_NB - 1) % 2).wait()


def kernel(tensor, image_size):
    rows, cols = tensor.shape
    rb = rows // _NB
    return pl.pallas_call(
        _body,
        in_specs=[pl.BlockSpec(memory_space=pl.ANY)],
        out_specs=pl.BlockSpec(memory_space=pl.ANY),
        out_shape=jax.ShapeDtypeStruct((rows, cols), tensor.dtype),
        scratch_shapes=[
            pltpu.VMEM((rows, cols), jnp.float32),
            pltpu.VMEM((2 * rb, cols), jnp.float32),
            pltpu.SemaphoreType.DMA((_NA,)),
            pltpu.SemaphoreType.DMA((2,)),
        ],
    )(tensor)


# speedup vs baseline: 1.0479x; 1.0479x over previous
import jax
import jax.numpy as jnp
from jax.experimental import pallas as pl
from jax.experimental.pallas import tpu as pltpu

_N_LEVELS = 2.0 ** 8 - 1.0
_NA = 4
_NB = 4


def _body(in_hbm, out_hbm, buf, ostage, isem, osem):
    rows, cols = buf.shape
    ra = rows // _NA
    rb = rows // _NB

    def copy_in(i, k):
        return pltpu.make_async_copy(
            in_hbm.at[pl.ds(i * ra, ra), :],
            buf.at[pl.ds(i * ra, ra), :],
            isem.at[k])

    def copy_out(i, k):
        return pltpu.make_async_copy(
            ostage.at[pl.ds(pl.multiple_of(k * rb, rb), rb), :],
            out_hbm.at[pl.ds(i * rb, rb), :],
            osem.at[k])

    for j in range(_NA):
        copy_in(j, j).start()

    def phase_a(i, m):
        copy_in(i, i).wait()
        band = buf[pl.ds(i * ra, ra), :]
        return jnp.maximum(m, jnp.max(jnp.abs(band)))

    alpha = jax.lax.fori_loop(0, _NA, phase_a, jnp.float32(0.0))

    d = jnp.maximum(2.0 * alpha, 1e-8)
    scale = _N_LEVELS / d
    zp = scale * (-alpha)
    inv = d * (1.0 / _N_LEVELS)

    def phase_b(i, carry):
        k = i % 2

        @pl.when(i >= 2)
        def _():
            copy_out(i - 2, k).wait()

        x = buf[pl.ds(i * rb, rb), :]
        q = jnp.round(x * scale - zp)
        ostage[pl.ds(pl.multiple_of(k * rb, rb), rb), :] = (q + zp) * inv
        copy_out(i, k).start()
        return carry

    jax.lax.fori_loop(0, _NB, phase_b, 0)
    copy_out(_NB - 2, (_NB - 2) % 2).wait()
    copy_out(_NB - 1, (_NB - 1) % 2).wait()


def kernel(tensor, image_size):
    rows, cols = tensor.shape
    rb = rows // _NB
    return pl.pallas_call(
        _body,
        in_specs=[pl.BlockSpec(memory_space=pl.ANY)],
        out_specs=pl.BlockSpec(memory_space=pl.ANY),
        out_shape=jax.ShapeDtypeStruct((rows, cols), tensor.dtype),
        scratch_shapes=[
            pltpu.VMEM((rows, cols), jnp.float32),
            pltpu.VMEM((2 * rb, cols), jnp.float32),
            pltpu.SemaphoreType.DMA((_NA,)),
            pltpu.SemaphoreType.DMA((2,)),
        ],
    )(tensor)


# speedup vs baseline: 1.0822x; 1.0328x over previous
import jax
import jax.numpy as jnp
from jax.experimental import pallas as pl
from jax.experimental.pallas import tpu as pltpu

_N_LEVELS = 2.0 ** 8 - 1.0
_NA = 8
_NB = 8


def _body(in_hbm, out_hbm, buf, isem, osem):
    rows, cols = buf.shape
    ra = rows // _NA
    rb = rows // _NB

    def copy_in(i):
        return pltpu.make_async_copy(
            in_hbm.at[pl.ds(i * ra, ra), :],
            buf.at[pl.ds(i * ra, ra), :],
            isem.at[i])

    def copy_out(i):
        return pltpu.make_async_copy(
            buf.at[pl.ds(i * rb, rb), :],
            out_hbm.at[pl.ds(i * rb, rb), :],
            osem.at[i])

    for j in range(_NA):
        copy_in(j).start()

    def phase_a(i, m):
        copy_in(i).wait()
        band = buf[pl.ds(i * ra, ra), :]
        return jnp.maximum(m, jnp.max(jnp.abs(band)))

    alpha = jax.lax.fori_loop(0, _NA, phase_a, jnp.float32(0.0))

    d = jnp.maximum(2.0 * alpha, 1e-8)
    scale = _N_LEVELS / d
    zp = scale * (-alpha)
    inv = d * (1.0 / _N_LEVELS)

    def phase_b(i, carry):
        x = buf[pl.ds(i * rb, rb), :]
        q = jnp.round(x * scale - zp)
        buf[pl.ds(i * rb, rb), :] = (q + zp) * inv
        copy_out(i).start()
        return carry

    jax.lax.fori_loop(0, _NB, phase_b, 0)
    for j in range(_NB):
        copy_out(j).wait()


def kernel(tensor, image_size):
    rows, cols = tensor.shape
    return pl.pallas_call(
        _body,
        in_specs=[pl.BlockSpec(memory_space=pl.ANY)],
        out_specs=pl.BlockSpec(memory_space=pl.ANY),
        out_shape=jax.ShapeDtypeStruct((rows, cols), tensor.dtype),
        scratch_shapes=[
            pltpu.VMEM((rows, cols), jnp.float32),
            pltpu.SemaphoreType.DMA((_NA,)),
            pltpu.SemaphoreType.DMA((_NB,)),
        ],
    )(tensor)


# speedup vs baseline: 1.0965x; 1.0132x over previous
import jax
import jax.numpy as jnp
from jax.experimental import pallas as pl
from jax.experimental.pallas import tpu as pltpu

_N_LEVELS = 2.0 ** 8 - 1.0
_NA = 8
_NB = 16


def _body(in_hbm, out_hbm, buf, isem, osem):
    rows, cols = buf.shape
    ra = rows // _NA
    rb = rows // _NB

    def copy_in(i):
        return pltpu.make_async_copy(
            in_hbm.at[pl.ds(i * ra, ra), :],
            buf.at[pl.ds(i * ra, ra), :],
            isem.at[i])

    def copy_out(i):
        return pltpu.make_async_copy(
            buf.at[pl.ds(i * rb, rb), :],
            out_hbm.at[pl.ds(i * rb, rb), :],
            osem.at[i])

    for j in range(_NA):
        copy_in(j).start()

    def phase_a(i, m):
        copy_in(i).wait()
        band = buf[pl.ds(i * ra, ra), :]
        return jnp.maximum(m, jnp.max(jnp.abs(band)))

    alpha = jax.lax.fori_loop(0, _NA, phase_a, jnp.float32(0.0))

    d = jnp.maximum(2.0 * alpha, 1e-8)
    scale = _N_LEVELS / d
    zp = scale * (-alpha)
    inv = d * (1.0 / _N_LEVELS)

    def phase_b(i, carry):
        x = buf[pl.ds(i * rb, rb), :]
        q = jnp.round(x * scale - zp)
        buf[pl.ds(i * rb, rb), :] = (q + zp) * inv
        copy_out(i).start()
        return carry

    jax.lax.fori_loop(0, _NB, phase_b, 0)
    for j in range(_NB):
        copy_out(j).wait()


def kernel(tensor, image_size):
    rows, cols = tensor.shape
    return pl.pallas_call(
        _body,
        in_specs=[pl.BlockSpec(memory_space=pl.ANY)],
        out_specs=pl.BlockSpec(memory_space=pl.ANY),
        out_shape=jax.ShapeDtypeStruct((rows, cols), tensor.dtype),
        scratch_shapes=[
            pltpu.VMEM((rows, cols), jnp.float32),
            pltpu.SemaphoreType.DMA((_NA,)),
            pltpu.SemaphoreType.DMA((_NB,)),
        ],
    )(tensor)
